# baseline (speedup 1.0000x reference)
import functools

import jax
import jax.numpy as jnp
from jax import lax
from jax.experimental import pallas as pl
from jax.experimental.pallas import tpu as pltpu
from jax.experimental.pallas import tpu_sc as plsc

N = 10000
E = 320000
C = 128
NC = 2
NS = 16
NW = NC * NS
CH = 80
EPW = E // NW
P = 5
CPA = 13
CPB = 12
NPAD = 10112
RPT = NPAD // NS
CH_H = 80
NCH_H = 125
NPAD_H = 10240
RPT_H = NPAD_H // NS

_mesh = plsc.VectorSubcoreMesh(core_axis_name="c", subcore_axis_name="s")


@functools.partial(
    pl.kernel,
    out_type=jax.ShapeDtypeStruct((NC, NPAD_H), jnp.float32),
    mesh=_mesh,
    scratch_types=[
        pltpu.VMEM((NCH_H, CH_H), jnp.int32),
        pltpu.VMEM((CH_H,), jnp.float32),
        pltpu.VMEM((RPT_H,), jnp.float32),
        pltpu.VMEM_SHARED((NPAD_H,), jnp.float32),
        pltpu.SemaphoreType.DMA,
    ],
)
def _sc_hist(dst_hbm, deg_hbm, idx_v, ones_v, zb_v, deg_sh, sem):
    c = lax.axis_index("c")
    s = lax.axis_index("s")
    wid = c * NS + s
    for i in range(CH_H // 16):
        ones_v[pl.ds(16 * i, 16)] = jnp.ones((16,), jnp.float32)

    def zinit(j, carry):
        zb_v[pl.ds(j * 16, 16)] = jnp.zeros((16,), jnp.float32)
        return carry

    lax.fori_loop(0, RPT_H // 16, zinit, 0)
    pltpu.sync_copy(zb_v, deg_sh.at[pl.ds(s * RPT_H, RPT_H)])
    plsc.subcore_barrier()
    pltpu.sync_copy(dst_hbm.at[wid], idx_v)

    def fire(j, carry):
        pltpu.async_copy(ones_v, deg_sh.at[idx_v.at[j]], sem, add=True)
        return carry

    lax.fori_loop(0, NCH_H, fire, 0)

    def drain(j, carry):
        pltpu.make_async_copy(ones_v, deg_sh.at[idx_v.at[0]], sem).wait()
        return carry

    lax.fori_loop(0, NCH_H, drain, 0)
    plsc.subcore_barrier()
    pltpu.sync_copy(deg_sh.at[pl.ds(s * RPT_H, RPT_H)],
                    deg_hbm.at[c, pl.ds(s * RPT_H, RPT_H)])


@functools.partial(
    pl.kernel,
    out_type=jax.ShapeDtypeStruct((NC, NPAD, C), jnp.float32),
    mesh=_mesh,
    scratch_types=[
        pltpu.VMEM((2, CPA, CH), jnp.int32),
        pltpu.VMEM((2, CPA, CH), jnp.int32),
        pltpu.VMEM((2, CH, C), jnp.float32),
        pltpu.VMEM_SHARED((NPAD, C), jnp.float32),
        pltpu.SemaphoreType.DMA,
        pltpu.SemaphoreType.DMA,
        pltpu.SemaphoreType.DMA,
        pltpu.SemaphoreType.DMA,
    ],
)
def _sc_scatter(y_hbm, src_hbm, dst_hbm, ap_hbm,
                src_v, dst_v, stg_v, a_sh, gs_a, gs_b, ss_a, ss_b):
    c = lax.axis_index("c")
    s = lax.axis_index("s")
    wid = c * NS + s

    def zinit(j, carry):
        for k in range(C // 16):
            stg_v[0, j, pl.ds(16 * k, 16)] = jnp.zeros((16,), jnp.float32)
        return carry

    lax.fori_loop(0, CH, zinit, 0)
    for k in range(RPT // CH):
        pltpu.sync_copy(stg_v.at[0], a_sh.at[pl.ds(s * RPT + k * CH, CH)])
    rem = RPT - (RPT // CH) * CH
    pltpu.sync_copy(stg_v.at[0, pl.ds(0, rem)],
                    a_sh.at[pl.ds(s * RPT + (RPT // CH) * CH, rem)])
    plsc.subcore_barrier()

    for p in range(P):
        pltpu.sync_copy(src_hbm.at[wid, p], src_v)
        pltpu.sync_copy(dst_hbm.at[wid, p], dst_v)
        pltpu.async_copy(y_hbm.at[src_v.at[0, 0]], stg_v.at[0], gs_a)
        pltpu.async_copy(y_hbm.at[src_v.at[1, 0]], stg_v.at[1], gs_b)

        def body(i, carry):
            for b, n_ch, gs in ((0, CPA, gs_a), (1, CPB, gs_b)):
                @pl.when(i < n_ch)
                def _():
                    pltpu.make_async_copy(
                        y_hbm.at[pl.ds(0, CH)], stg_v.at[b], gs).wait()
                    pltpu.async_copy(
                        stg_v.at[b], a_sh.at[dst_v.at[b, i]],
                        ss_a if b == 0 else ss_b, add=True)

            for b, n_ch, gs, ss in ((0, CPA, gs_a, ss_a),
                                    (1, CPB, gs_b, ss_b)):
                @pl.when(i < n_ch)
                def _():
                    pltpu.make_async_copy(
                        y_hbm.at[pl.ds(0, CH)], stg_v.at[b], ss).wait()

                    @pl.when(i + 1 < n_ch)
                    def _():
                        pltpu.async_copy(y_hbm.at[src_v.at[b, i + 1]],
                                         stg_v.at[b], gs)

            return carry

        lax.fori_loop(0, CPA, body, 0)
    plsc.subcore_barrier()
    pltpu.sync_copy(a_sh.at[pl.ds(s * RPT, RPT)],
                    ap_hbm.at[c, pl.ds(s * RPT, RPT)])


def _scale_body(deg_ref, xx_ref, y_ref):
    deg = deg_ref[:, 0:1] + deg_ref[:, 1:2] + 1.0
    y_ref[...] = xx_ref[...] * lax.rsqrt(deg)


def _tc_scale(xx, degt):
    rb = 1000
    return pl.pallas_call(
        _scale_body,
        grid=(N // rb,),
        in_specs=[
            pl.BlockSpec((rb, NC), lambda i: (i, 0)),
            pl.BlockSpec((rb, C), lambda i: (i, 0)),
        ],
        out_specs=pl.BlockSpec((rb, C), lambda i: (i, 0)),
        out_shape=jax.ShapeDtypeStruct((N, C), jnp.float32),
    )(degt, xx)


def _dense_body(ap_ref, y_ref, xx_ref, deg_ref, w_ref, w1_ref, b1_ref,
                w2_ref, b2_ref, out_ref):
    deg = deg_ref[:, 0:1] + deg_ref[:, 1:2] + 1.0
    dinv = lax.rsqrt(deg)
    a = (ap_ref[0] + ap_ref[1] + y_ref[...]) * dinv
    ww1 = jnp.dot(w_ref[...], w1_ref[...], preferred_element_type=jnp.float32)
    h = jnp.maximum(
        jnp.dot(a, ww1, preferred_element_type=jnp.float32) + b1_ref[...], 0.0)
    up = jnp.dot(h, w2_ref[...], preferred_element_type=jnp.float32) + b2_ref[...]
    out_ref[...] = xx_ref[...] + up


def _tc_dense(ap, y, xx, degt, W, W1, b1, W2, b2):
    rb = 1000
    return pl.pallas_call(
        _dense_body,
        grid=(N // rb,),
        in_specs=[
            pl.BlockSpec((NC, rb, C), lambda i: (0, i, 0)),
            pl.BlockSpec((rb, C), lambda i: (i, 0)),
            pl.BlockSpec((rb, C), lambda i: (i, 0)),
            pl.BlockSpec((rb, NC), lambda i: (i, 0)),
            pl.BlockSpec((C, 3 * C), lambda i: (0, 0)),
            pl.BlockSpec((3 * C, 32), lambda i: (0, 0)),
            pl.BlockSpec((1, 32), lambda i: (0, 0)),
            pl.BlockSpec((32, C), lambda i: (0, 0)),
            pl.BlockSpec((1, C), lambda i: (0, 0)),
        ],
        out_specs=pl.BlockSpec((rb, C), lambda i: (i, 0)),
        out_shape=jax.ShapeDtypeStruct((N, C), jnp.float32),
    )(ap, y, xx, degt, W, W1, b1, W2, b2)


def kernel(xx, edge_index, parent_index, W, W1, b1, W2, b2):
    ei = edge_index.astype(jnp.int32)

    def _arrange(flat):
        e5 = flat.reshape(NW, P, CPA + CPB, CH)
        a_half = e5[:, :, :CPA]
        b_half = jnp.concatenate(
            [e5[:, :, CPA:], jnp.zeros((NW, P, 1, CH), jnp.int32)], axis=2)
        return jnp.stack([a_half, b_half], axis=2)

    src_r = _arrange(ei[0])
    dst_r = _arrange(ei[1])
    degp = _sc_hist(ei[1].reshape(NW, NCH_H, CH_H))
    degt = degp.T
    y = _tc_scale(xx, degt)
    ap = _sc_scatter(y, src_r, dst_r)
    return _tc_dense(ap, y, xx, degt, W, W1,
                     b1.reshape(1, -1), W2, b2.reshape(1, -1))


# speedup vs baseline: 1.2787x; 1.2787x over previous
import functools

import jax
import jax.numpy as jnp
from jax import lax
from jax.experimental import pallas as pl
from jax.experimental.pallas import tpu as pltpu
from jax.experimental.pallas import tpu_sc as plsc

N = 10000
E = 320000
C = 128
NC = 2
NS = 16
NW = NC * NS
CH = 128
EPW = E // NW
P = 3
CPA = 13
CPB = 13
ER = EPW - P * (CPA + CPB) * CH
NPAD = 10112
RPT = NPAD // NS
CH_H = 80
NCH_H = 125
NPAD_H = 10240
RPT_H = NPAD_H // NS

_mesh = plsc.VectorSubcoreMesh(core_axis_name="c", subcore_axis_name="s")


@functools.partial(
    pl.kernel,
    out_type=jax.ShapeDtypeStruct((NC, NPAD_H), jnp.float32),
    mesh=_mesh,
    scratch_types=[
        pltpu.VMEM((NCH_H, CH_H), jnp.int32),
        pltpu.VMEM((CH_H,), jnp.float32),
        pltpu.VMEM((RPT_H,), jnp.float32),
        pltpu.VMEM_SHARED((NPAD_H,), jnp.float32),
        pltpu.SemaphoreType.DMA,
    ],
)
def _sc_hist(dst_hbm, deg_hbm, idx_v, ones_v, zb_v, deg_sh, sem):
    c = lax.axis_index("c")
    s = lax.axis_index("s")
    wid = c * NS + s
    for i in range(CH_H // 16):
        ones_v[pl.ds(16 * i, 16)] = jnp.ones((16,), jnp.float32)

    def zinit(j, carry):
        zb_v[pl.ds(j * 16, 16)] = jnp.zeros((16,), jnp.float32)
        return carry

    lax.fori_loop(0, RPT_H // 16, zinit, 0)
    pltpu.sync_copy(zb_v, deg_sh.at[pl.ds(s * RPT_H, RPT_H)])
    plsc.subcore_barrier()
    pltpu.sync_copy(dst_hbm.at[wid], idx_v)

    def fire(j, carry):
        pltpu.async_copy(ones_v, deg_sh.at[idx_v.at[j]], sem, add=True)
        return carry

    lax.fori_loop(0, NCH_H, fire, 0)

    def drain(j, carry):
        pltpu.make_async_copy(ones_v, deg_sh.at[idx_v.at[0]], sem).wait()
        return carry

    lax.fori_loop(0, NCH_H, drain, 0)
    plsc.subcore_barrier()
    pltpu.sync_copy(deg_sh.at[pl.ds(s * RPT_H, RPT_H)],
                    deg_hbm.at[c, pl.ds(s * RPT_H, RPT_H)])


@functools.partial(
    pl.kernel,
    out_type=jax.ShapeDtypeStruct((NC, NPAD, C), jnp.float32),
    mesh=_mesh,
    scratch_types=[
        pltpu.VMEM((2, CPA, CH), jnp.int32),
        pltpu.VMEM((2, CPA, CH), jnp.int32),
        pltpu.VMEM((2, CH, C), jnp.float32),
        pltpu.VMEM((ER,), jnp.int32),
        pltpu.VMEM((ER,), jnp.int32),
        pltpu.VMEM_SHARED((NPAD, C), jnp.float32),
        pltpu.SemaphoreType.DMA,
        pltpu.SemaphoreType.DMA,
        pltpu.SemaphoreType.DMA,
        pltpu.SemaphoreType.DMA,
    ],
)
def _sc_scatter(y_hbm, src_hbm, dst_hbm, srce_hbm, dste_hbm, ap_hbm,
                src_v, dst_v, stg_v, se_v, de_v, a_sh,
                gs_a, gs_b, ss_a, ss_b):
    c = lax.axis_index("c")
    s = lax.axis_index("s")
    wid = c * NS + s

    def zinit(j, carry):
        for k in range(C // 16):
            stg_v[0, j, pl.ds(16 * k, 16)] = jnp.zeros((16,), jnp.float32)
        return carry

    lax.fori_loop(0, CH, zinit, 0)
    for k in range(RPT // CH):
        pltpu.sync_copy(stg_v.at[0], a_sh.at[pl.ds(s * RPT + k * CH, CH)])
    rem = RPT - (RPT // CH) * CH
    pltpu.sync_copy(stg_v.at[0, pl.ds(0, rem)],
                    a_sh.at[pl.ds(s * RPT + (RPT // CH) * CH, rem)])
    plsc.subcore_barrier()

    for p in range(P):
        pltpu.sync_copy(src_hbm.at[wid, p], src_v)
        pltpu.sync_copy(dst_hbm.at[wid, p], dst_v)
        pltpu.async_copy(y_hbm.at[src_v.at[0, 0]], stg_v.at[0], gs_a)
        pltpu.async_copy(y_hbm.at[src_v.at[1, 0]], stg_v.at[1], gs_b)

        def body(i, carry):
            for b, n_ch, gs, ss in ((0, CPA, gs_a, ss_a),
                                    (1, CPB, gs_b, ss_b)):
                @pl.when(i < n_ch)
                def _():
                    pltpu.make_async_copy(
                        y_hbm.at[pl.ds(0, CH)], stg_v.at[b], gs).wait()
                    pltpu.async_copy(
                        stg_v.at[b], a_sh.at[dst_v.at[b, i]], ss, add=True)
                    pltpu.make_async_copy(
                        y_hbm.at[pl.ds(0, CH)], stg_v.at[b], ss).wait()

                    @pl.when(i + 1 < n_ch)
                    def _():
                        pltpu.async_copy(y_hbm.at[src_v.at[b, i + 1]],
                                         stg_v.at[b], gs)

            return carry

        lax.fori_loop(0, CPA, body, 0)

    pltpu.sync_copy(srce_hbm.at[wid], se_v)
    pltpu.sync_copy(dste_hbm.at[wid], de_v)
    pltpu.async_copy(y_hbm.at[se_v], stg_v.at[0, pl.ds(0, ER)], gs_a).wait()
    pltpu.sync_copy(stg_v.at[0, pl.ds(0, ER)], a_sh.at[de_v], add=True)
    plsc.subcore_barrier()
    pltpu.sync_copy(a_sh.at[pl.ds(s * RPT, RPT)],
                    ap_hbm.at[c, pl.ds(s * RPT, RPT)])


def _scale_body(deg_ref, xx_ref, y_ref):
    deg = deg_ref[:, 0:1] + deg_ref[:, 1:2] + 1.0
    y_ref[...] = xx_ref[...] * lax.rsqrt(deg)


def _tc_scale(xx, degt):
    rb = 1000
    return pl.pallas_call(
        _scale_body,
        grid=(N // rb,),
        in_specs=[
            pl.BlockSpec((rb, NC), lambda i: (i, 0)),
            pl.BlockSpec((rb, C), lambda i: (i, 0)),
        ],
        out_specs=pl.BlockSpec((rb, C), lambda i: (i, 0)),
        out_shape=jax.ShapeDtypeStruct((N, C), jnp.float32),
    )(degt, xx)


def _dense_body(ap_ref, y_ref, xx_ref, deg_ref, w_ref, w1_ref, b1_ref,
                w2_ref, b2_ref, out_ref):
    deg = deg_ref[:, 0:1] + deg_ref[:, 1:2] + 1.0
    dinv = lax.rsqrt(deg)
    a = (ap_ref[0] + ap_ref[1] + y_ref[...]) * dinv
    ww1 = jnp.dot(w_ref[...], w1_ref[...], preferred_element_type=jnp.float32)
    h = jnp.maximum(
        jnp.dot(a, ww1, preferred_element_type=jnp.float32) + b1_ref[...], 0.0)
    up = jnp.dot(h, w2_ref[...], preferred_element_type=jnp.float32) + b2_ref[...]
    out_ref[...] = xx_ref[...] + up


def _tc_dense(ap, y, xx, degt, W, W1, b1, W2, b2):
    rb = 1000
    return pl.pallas_call(
        _dense_body,
        grid=(N // rb,),
        in_specs=[
            pl.BlockSpec((NC, rb, C), lambda i: (0, i, 0)),
            pl.BlockSpec((rb, C), lambda i: (i, 0)),
            pl.BlockSpec((rb, C), lambda i: (i, 0)),
            pl.BlockSpec((rb, NC), lambda i: (i, 0)),
            pl.BlockSpec((C, 3 * C), lambda i: (0, 0)),
            pl.BlockSpec((3 * C, 32), lambda i: (0, 0)),
            pl.BlockSpec((1, 32), lambda i: (0, 0)),
            pl.BlockSpec((32, C), lambda i: (0, 0)),
            pl.BlockSpec((1, C), lambda i: (0, 0)),
        ],
        out_specs=pl.BlockSpec((rb, C), lambda i: (i, 0)),
        out_shape=jax.ShapeDtypeStruct((N, C), jnp.float32),
    )(ap, y, xx, degt, W, W1, b1, W2, b2)


def kernel(xx, edge_index, parent_index, W, W1, b1, W2, b2):
    ei = edge_index.astype(jnp.int32)

    nfull = P * (CPA + CPB) * CH

    def _arrange(flat):
        e = flat.reshape(NW, EPW)
        full = e[:, :nfull].reshape(NW, P, 2, CPA, CH)
        return full, e[:, nfull:]

    src_r, src_e = _arrange(ei[0])
    dst_r, dst_e = _arrange(ei[1])
    degp = _sc_hist(ei[1].reshape(NW, NCH_H, CH_H))
    degt = degp.T
    y = _tc_scale(xx, degt)
    ap = _sc_scatter(y, src_r, dst_r, src_e, dst_e)
    return _tc_dense(ap, y, xx, degt, W, W1,
                     b1.reshape(1, -1), W2, b2.reshape(1, -1))
